# baseline (speedup 1.0000x reference)
import functools

import jax
import jax.numpy as jnp
from jax import lax
from jax.experimental import pallas as pl
from jax.experimental.pallas import tpu as pltpu
from jax.experimental.pallas import tpu_sc as plsc

NSEG = 512
D = 128
N = 100000
NC = 2
NS = 16
NW = NC * NS
L = 16
R = 80
NCHUNK = N // R
CPT = NCHUNK // NW
NLEFT = NCHUNK - CPT * NW
NSLOT = CPT + 1
GB = 4
NGROUP = NSLOT // GB
NBUF = 2 * GB
NSTAGE = 16
DUMMY = NSEG
NSEG_PAD = NSEG + 8
SEG_PER_TILE = NSEG // NS
RU = 4
NJ = D // L


def _sc_body(nodes_hbm, batch_hbm, part_hbm,
             rows_v, idx_v, buf_v, stage_v, sidx_v, acc_sh, *sems):
  load_sems = sems[:NBUF]
  idx_sems = sems[NBUF:2 * NBUF]
  scat_sems = sems[2 * NBUF:]
  c = lax.axis_index("c")
  s = lax.axis_index("s")
  w = c * NS + s
  nsl = jnp.where(w < NLEFT, NSLOT, NSLOT - 1)

  zero16 = jnp.zeros((L,), jnp.float32)
  dummy16 = jnp.full((L,), DUMMY, jnp.int32)
  lane_iota = lax.iota(jnp.int32, L)

  def chunk_of(slot):
    return jnp.where(slot == CPT, NW * CPT + w, w * CPT + slot)

  def fire_loads(slot, buf):
    @pl.when(slot < nsl)
    def _():
      ch = chunk_of(slot)
      pltpu.async_copy(batch_hbm.at[pl.ds(ch * R, R)], idx_v.at[buf],
                       idx_sems[buf])
      pltpu.async_copy(nodes_hbm.at[pl.ds(ch * R, R)], rows_v.at[buf],
                       load_sems[buf])

  def wait_loads(slot, buf):
    @pl.when(slot < nsl)
    def _():
      ch = chunk_of(slot)
      pltpu.make_async_copy(batch_hbm.at[pl.ds(ch * R, R)], idx_v.at[buf],
                            idx_sems[buf]).wait()
      pltpu.make_async_copy(nodes_hbm.at[pl.ds(ch * R, R)], rows_v.at[buf],
                            load_sems[buf]).wait()

  for slot in range(2 * GB):
    fire_loads(slot, slot)

  for i in range(SEG_PER_TILE):
    for j in range(NJ):
      buf_v[i, pl.ds(j * L, L)] = zero16
  pltpu.sync_copy(buf_v, acc_sh.at[pl.ds(s * SEG_PER_TILE, SEG_PER_TILE)])

  sidx_v[...] = dummy16

  plsc.subcore_barrier()

  def group_proc(k, eo, pos):
    scat_conds = [None] * GB
    for b in range(GB):
      slot = (2 * k + eo) * GB + b
      buf = eo * GB + b
      valid = slot < nsl

      flush = pos > NSTAGE - 2

      @pl.when(flush)
      def _():
        pltpu.sync_copy(stage_v, acc_sh.at[sidx_v], add=True)
        sidx_v[...] = dummy16

      pos = jnp.where(flush, 0, pos)

      wait_loads(slot, buf)

      lo = idx_v[buf, pl.ds(0, L)][0]
      hi = idx_v[buf, pl.ds(R - L, L)][L - 1]
      single = lo == hi
      okv = jnp.logical_and(single, valid)

      @pl.when(okv)
      def _():
        def row_body(i, rs):
          out = list(rs)
          for r in range(RU):
            ridx = i * RU + r
            for j in range(NJ):
              out[j] = out[j] + rows_v[buf, ridx, pl.ds(j * L, L)]
          return tuple(out)

        rowsum = lax.fori_loop(0, R // RU, row_body, (zero16,) * NJ)
        for j in range(NJ):
          stage_v[pos, pl.ds(j * L, L)] = rowsum[j]
        cur = sidx_v[...]
        sidx_v[...] = jnp.where(lane_iota == pos, lo, cur)

      scat_conds[b] = jnp.logical_and(valid, jnp.logical_not(single))

      @pl.when(scat_conds[b])
      def _():
        pltpu.async_copy(rows_v.at[buf], acc_sh.at[idx_v.at[buf]],
                         scat_sems[buf], add=True)

      pos = pos + jnp.where(okv, 1, 0).astype(jnp.int32)

    for b in range(GB):
      buf = eo * GB + b

      @pl.when(scat_conds[b])
      def _(buf=buf):
        pltpu.make_async_copy(rows_v.at[buf], acc_sh.at[idx_v.at[buf]],
                              scat_sems[buf]).wait()

      fire_loads((2 * k + eo + 2) * GB + b, buf)

    return pos

  def outer(k, pos):
    return group_proc(k, 1, group_proc(k, 0, pos))

  lax.fori_loop(0, NGROUP // 2, outer, jnp.int32(0))

  pltpu.sync_copy(stage_v, acc_sh.at[sidx_v], add=True)

  plsc.subcore_barrier()

  pltpu.sync_copy(acc_sh.at[pl.ds(s * SEG_PER_TILE, SEG_PER_TILE)], buf_v)
  pltpu.sync_copy(buf_v, part_hbm.at[c, pl.ds(s * SEG_PER_TILE, SEG_PER_TILE)])


def _sc_segment_sum(nodes, batch):
  mesh = plsc.VectorSubcoreMesh(core_axis_name="c", subcore_axis_name="s")
  return pl.kernel(
      _sc_body,
      out_type=jax.ShapeDtypeStruct((NC, NSEG, D), jnp.float32),
      mesh=mesh,
      scratch_types=[
          pltpu.VMEM((NBUF, R, D), jnp.float32),
          pltpu.VMEM((NBUF, R), jnp.int32),
          pltpu.VMEM((SEG_PER_TILE, D), jnp.float32),
          pltpu.VMEM((NSTAGE, D), jnp.float32),
          pltpu.VMEM((NSTAGE,), jnp.int32),
          pltpu.VMEM_SHARED((NSEG_PAD, D), jnp.float32),
      ] + [pltpu.SemaphoreType.DMA] * (3 * NBUF),
  )(nodes, batch)


def _combine_body(p_ref, o_ref):
  o_ref[...] = p_ref[0] + p_ref[1]


def _combine(partials):
  return pl.pallas_call(
      _combine_body,
      out_shape=jax.ShapeDtypeStruct((NSEG, D), jnp.float32),
  )(partials)


@jax.jit
def _run(nodes, batch):
  partials = _sc_segment_sum(nodes, batch.astype(jnp.int32))
  return _combine(partials)


def kernel(nodes, batch):
  return _run(nodes, batch)


# speedup vs baseline: 1.0086x; 1.0086x over previous
import functools

import jax
import jax.numpy as jnp
from jax import lax
from jax.experimental import pallas as pl
from jax.experimental.pallas import tpu as pltpu
from jax.experimental.pallas import tpu_sc as plsc

NSEG = 512
D = 128
N = 100000
NC = 2
NS = 16
NW = NC * NS
L = 16
R = 80
NCHUNK = N // R
CPT = NCHUNK // NW
NLEFT = NCHUNK - CPT * NW
NSLOT = CPT + 1
GB = 4
NGROUP = NSLOT // GB
NBUF = 2 * GB
NSTAGE = 16
DUMMY = NSEG
NSEG_PAD = NSEG + 8
SEG_PER_TILE = NSEG // NS
RU = 2
NJ = D // L


def _sc_body(nodes_hbm, batch_hbm, part_hbm,
             rows_v, idx_v, buf_v, stage_v, sidx_v, acc_sh, *sems):
  load_sems = sems[:NBUF]
  idx_sems = sems[NBUF:2 * NBUF]
  scat_sems = sems[2 * NBUF:]
  c = lax.axis_index("c")
  s = lax.axis_index("s")
  w = c * NS + s
  nsl = jnp.where(w < NLEFT, NSLOT, NSLOT - 1)

  zero16 = jnp.zeros((L,), jnp.float32)
  dummy16 = jnp.full((L,), DUMMY, jnp.int32)
  lane_iota = lax.iota(jnp.int32, L)

  def chunk_of(slot):
    return jnp.where(slot == CPT, NW * CPT + w, w * CPT + slot)

  def fire_loads(slot, buf):
    @pl.when(slot < nsl)
    def _():
      ch = chunk_of(slot)
      pltpu.async_copy(batch_hbm.at[pl.ds(ch * R, R)], idx_v.at[buf],
                       idx_sems[buf])
      pltpu.async_copy(nodes_hbm.at[pl.ds(ch * R, R)], rows_v.at[buf],
                       load_sems[buf])

  def wait_loads(slot, buf):
    @pl.when(slot < nsl)
    def _():
      ch = chunk_of(slot)
      pltpu.make_async_copy(batch_hbm.at[pl.ds(ch * R, R)], idx_v.at[buf],
                            idx_sems[buf]).wait()
      pltpu.make_async_copy(nodes_hbm.at[pl.ds(ch * R, R)], rows_v.at[buf],
                            load_sems[buf]).wait()

  for slot in range(2 * GB):
    fire_loads(slot, slot)

  for i in range(SEG_PER_TILE):
    for j in range(NJ):
      buf_v[i, pl.ds(j * L, L)] = zero16
  pltpu.sync_copy(buf_v, acc_sh.at[pl.ds(s * SEG_PER_TILE, SEG_PER_TILE)])

  sidx_v[...] = dummy16

  plsc.subcore_barrier()

  def group_proc(k, eo, pos):
    scat_conds = [None] * GB
    for b in range(GB):
      slot = (2 * k + eo) * GB + b
      buf = eo * GB + b
      valid = slot < nsl

      flush = pos > NSTAGE - 2

      @pl.when(flush)
      def _():
        pltpu.sync_copy(stage_v, acc_sh.at[sidx_v], add=True)
        sidx_v[...] = dummy16

      pos = jnp.where(flush, 0, pos)

      wait_loads(slot, buf)

      lo = idx_v[buf, pl.ds(0, L)][0]
      hi = idx_v[buf, pl.ds(R - L, L)][L - 1]
      single = lo == hi
      okv = jnp.logical_and(single, valid)

      @pl.when(okv)
      def _():
        def row_body(i, rs):
          out = list(rs)
          for r in range(RU):
            ridx = i * RU + r
            for j in range(NJ):
              out[j] = out[j] + rows_v[buf, ridx, pl.ds(j * L, L)]
          return tuple(out)

        rowsum = lax.fori_loop(0, R // RU, row_body, (zero16,) * NJ)
        for j in range(NJ):
          stage_v[pos, pl.ds(j * L, L)] = rowsum[j]
        cur = sidx_v[...]
        sidx_v[...] = jnp.where(lane_iota == pos, lo, cur)

      scat_conds[b] = jnp.logical_and(valid, jnp.logical_not(single))

      @pl.when(scat_conds[b])
      def _():
        pltpu.async_copy(rows_v.at[buf], acc_sh.at[idx_v.at[buf]],
                         scat_sems[buf], add=True)

      pos = pos + jnp.where(okv, 1, 0).astype(jnp.int32)

    for b in range(GB):
      buf = eo * GB + b

      @pl.when(scat_conds[b])
      def _(buf=buf):
        pltpu.make_async_copy(rows_v.at[buf], acc_sh.at[idx_v.at[buf]],
                              scat_sems[buf]).wait()

      fire_loads((2 * k + eo + 2) * GB + b, buf)

    return pos

  def outer(k, pos):
    return group_proc(k, 1, group_proc(k, 0, pos))

  lax.fori_loop(0, NGROUP // 2, outer, jnp.int32(0))

  pltpu.sync_copy(stage_v, acc_sh.at[sidx_v], add=True)

  plsc.subcore_barrier()

  pltpu.sync_copy(acc_sh.at[pl.ds(s * SEG_PER_TILE, SEG_PER_TILE)], buf_v)
  pltpu.sync_copy(buf_v, part_hbm.at[c, pl.ds(s * SEG_PER_TILE, SEG_PER_TILE)])


def _sc_segment_sum(nodes, batch):
  mesh = plsc.VectorSubcoreMesh(core_axis_name="c", subcore_axis_name="s")
  return pl.kernel(
      _sc_body,
      out_type=jax.ShapeDtypeStruct((NC, NSEG, D), jnp.float32),
      mesh=mesh,
      scratch_types=[
          pltpu.VMEM((NBUF, R, D), jnp.float32),
          pltpu.VMEM((NBUF, R), jnp.int32),
          pltpu.VMEM((SEG_PER_TILE, D), jnp.float32),
          pltpu.VMEM((NSTAGE, D), jnp.float32),
          pltpu.VMEM((NSTAGE,), jnp.int32),
          pltpu.VMEM_SHARED((NSEG_PAD, D), jnp.float32),
      ] + [pltpu.SemaphoreType.DMA] * (3 * NBUF),
  )(nodes, batch)


def _combine_body(p_ref, o_ref):
  o_ref[...] = p_ref[0] + p_ref[1]


def _combine(partials):
  return pl.pallas_call(
      _combine_body,
      out_shape=jax.ShapeDtypeStruct((NSEG, D), jnp.float32),
  )(partials)


@jax.jit
def _run(nodes, batch):
  partials = _sc_segment_sum(nodes, batch.astype(jnp.int32))
  return _combine(partials)


def kernel(nodes, batch):
  return _run(nodes, batch)
